# baseline (speedup 1.0000x reference)
import functools

import jax
import jax.numpy as jnp
from jax.experimental import pallas as pl
from jax.experimental.pallas import tpu as pltpu


def _round_up(x: int, m: int) -> int:
    return (x + m - 1) // m * m


def _relayout_kernel(src_ref, dst_ref):
    vc, e = src_ref.shape
    dst_ref[...] = src_ref[...].astype(jnp.bfloat16).reshape(vc, 1, e)


def _dan_kernel(ids_ref,
                table_ref,
                w1_ref,
                b1_ref,
                w2_ref,
                b2_ref,
                out_ref,
                pooled_ref,
                *, tile_b: int, seq_len: int):
    base = pl.program_id(0) * (tile_b * seq_len)
    inv_s = jnp.float32(1.0 / seq_len)

    nacc = min(2, seq_len)

    def row_f32(idx):
        return table_ref[idx, 0].astype(jnp.float32)

    @pl.loop(0, tile_b)
    def _(b):
        row = base + b * seq_len
        accs = [row_f32(ids_ref[row + j]) for j in range(nacc)]
        for s in range(nacc, seq_len):
            j = s % nacc
            accs[j] = accs[j] + row_f32(ids_ref[row + s])
        while len(accs) > 1:
            accs = [a + b2 for a, b2 in zip(accs[0::2], accs[1::2])] + (
                [accs[-1]] if len(accs) % 2 else [])
        pooled_ref[b, :] = accs[0] * inv_s

    h = jnp.dot(pooled_ref[...], w1_ref[...],
                preferred_element_type=jnp.float32) + b1_ref[...]
    h = jnp.maximum(h, 0.0)

    logits = jnp.dot(h, w2_ref[...],
                     preferred_element_type=jnp.float32) + b2_ref[...]

    m = jnp.max(logits, axis=1, keepdims=True)
    lse = m + jnp.log(jnp.sum(jnp.exp(logits - m), axis=1, keepdims=True))
    out_ref[...] = logits - lse


def kernel(token_ids, emb_table, w1, b1, w2, b2):
    B, S = token_ids.shape
    V, E = emb_table.shape
    H = w1.shape[1]
    C = w2.shape[1]

    TB = 128 if B >= 128 else _round_up(max(B, 8), 8)
    B_pad = _round_up(B, TB)
    E_pad = _round_up(max(E, 128), 128)
    H_pad = _round_up(max(H, 128), 128)
    C_pad = _round_up(max(C, 128), 128)

    ids = token_ids.astype(jnp.int32)
    if B_pad != B:
        ids = jnp.pad(ids, ((0, B_pad - B), (0, 0)))
    ids_flat = ids.reshape(B_pad * S)

    V_pad = _round_up(V, 8)
    table = emb_table.astype(jnp.float32)
    if E_pad != E or V_pad != V:
        table = jnp.pad(table, ((0, V_pad - V), (0, E_pad - E)))

    nchunk = 16
    while nchunk > 1 and V_pad % (nchunk * 8) != 0:
        nchunk //= 2
    vc = V_pad // nchunk
    table3 = pl.pallas_call(
        _relayout_kernel,
        out_shape=jax.ShapeDtypeStruct((V_pad, 1, E_pad), jnp.bfloat16),
        grid=(nchunk,),
        in_specs=[pl.BlockSpec((vc, E_pad), lambda i: (i, 0))],
        out_specs=pl.BlockSpec((vc, 1, E_pad), lambda i: (i, 0, 0)),
        compiler_params=pltpu.CompilerParams(
            dimension_semantics=("parallel",),
        ),
    )(table)

    w1_p = w1.astype(jnp.float32)
    if (E_pad, H_pad) != (E, H):
        w1_p = jnp.pad(w1_p, ((0, E_pad - E), (0, H_pad - H)))
    b1_p = b1.astype(jnp.float32).reshape(1, H)
    if H_pad != H:
        b1_p = jnp.pad(b1_p, ((0, 0), (0, H_pad - H)))
    w2_p = w2.astype(jnp.float32)
    if (H_pad, C_pad) != (H, C):
        w2_p = jnp.pad(w2_p, ((0, H_pad - H), (0, C_pad - C)))
    b2_p = b2.astype(jnp.float32).reshape(1, C)
    if C_pad != C:
        b2_p = jnp.pad(b2_p, ((0, 0), (0, C_pad - C)),
                       constant_values=-1e30)

    body = functools.partial(_dan_kernel, tile_b=TB, seq_len=S)

    out = pl.pallas_call(
        body,
        out_shape=jax.ShapeDtypeStruct((B_pad, C_pad), jnp.float32),
        grid_spec=pltpu.PrefetchScalarGridSpec(
            num_scalar_prefetch=1,
            grid=(B_pad // TB,),
            in_specs=[
                pl.BlockSpec((V_pad, 1, E_pad), lambda i, ids: (0, 0, 0)),
                pl.BlockSpec((E_pad, H_pad), lambda i, ids: (0, 0)),
                pl.BlockSpec((1, H_pad), lambda i, ids: (0, 0)),
                pl.BlockSpec((H_pad, C_pad), lambda i, ids: (0, 0)),
                pl.BlockSpec((1, C_pad), lambda i, ids: (0, 0)),
            ],
            out_specs=pl.BlockSpec((TB, C_pad), lambda i, ids: (i, 0)),
            scratch_shapes=[pltpu.VMEM((TB, E_pad), jnp.float32)],
        ),
        compiler_params=pltpu.CompilerParams(
            dimension_semantics=("parallel",),
            vmem_limit_bytes=48 * 1024 * 1024,
        ),
    )(ids_flat, table3, w1_p, b1_p, w2_p, b2_p)

    if B_pad != B or C_pad != C:
        out = out[:B, :C]
    return out


# speedup vs baseline: 1.2770x; 1.2770x over previous
import functools

import jax
import jax.numpy as jnp
from jax.experimental import pallas as pl
from jax.experimental.pallas import tpu as pltpu


def _round_up(x: int, m: int) -> int:
    return (x + m - 1) // m * m


def _dan_kernel(ids_ref,
                table_hbm,
                w1_ref,
                b1_ref,
                w2_ref,
                b2_ref,
                out_ref,
                table_ref,
                pooled_ref,
                sem,
                *, tile_b: int, seq_len: int):
    @pl.when(pl.program_id(0) == 0)
    def _():
        pltpu.make_async_copy(table_hbm, table_ref.at[:, 0], sem).start()
        pltpu.make_async_copy(table_hbm, table_ref.at[:, 0], sem).wait()

    base = pl.program_id(0) * (tile_b * seq_len)
    inv_s = jnp.float32(1.0 / seq_len)

    nacc = min(2, seq_len)

    @pl.loop(0, tile_b)
    def _(b):
        row = base + b * seq_len
        accs = [table_ref[ids_ref[row + j], 0] for j in range(nacc)]
        for s in range(nacc, seq_len):
            j = s % nacc
            accs[j] = accs[j] + table_ref[ids_ref[row + s], 0]
        while len(accs) > 1:
            accs = [a + b2 for a, b2 in zip(accs[0::2], accs[1::2])] + (
                [accs[-1]] if len(accs) % 2 else [])
        pooled_ref[b, :] = accs[0] * inv_s

    h = jnp.dot(pooled_ref[...], w1_ref[...],
                preferred_element_type=jnp.float32) + b1_ref[...]
    h = jnp.maximum(h, 0.0)

    logits = jnp.dot(h, w2_ref[...],
                     preferred_element_type=jnp.float32) + b2_ref[...]

    m = jnp.max(logits, axis=1, keepdims=True)
    lse = m + jnp.log(jnp.sum(jnp.exp(logits - m), axis=1, keepdims=True))
    out_ref[...] = logits - lse


def kernel(token_ids, emb_table, w1, b1, w2, b2):
    B, S = token_ids.shape
    V, E = emb_table.shape
    H = w1.shape[1]
    C = w2.shape[1]

    TB = 128 if B >= 128 else _round_up(max(B, 8), 8)
    B_pad = _round_up(B, TB)
    E_pad = _round_up(max(E, 128), 128)
    H_pad = _round_up(max(H, 128), 128)
    C_pad = _round_up(max(C, 128), 128)

    ids = token_ids.astype(jnp.int32)
    if B_pad != B:
        ids = jnp.pad(ids, ((0, B_pad - B), (0, 0)))
    ids_flat = ids.reshape(B_pad * S)

    V_pad = _round_up(V, 8)
    table = emb_table.astype(jnp.float32)
    if E_pad != E or V_pad != V:
        table = jnp.pad(table, ((0, V_pad - V), (0, E_pad - E)))

    w1_p = w1.astype(jnp.float32)
    if (E_pad, H_pad) != (E, H):
        w1_p = jnp.pad(w1_p, ((0, E_pad - E), (0, H_pad - H)))
    b1_p = b1.astype(jnp.float32).reshape(1, H)
    if H_pad != H:
        b1_p = jnp.pad(b1_p, ((0, 0), (0, H_pad - H)))
    w2_p = w2.astype(jnp.float32)
    if (H_pad, C_pad) != (H, C):
        w2_p = jnp.pad(w2_p, ((0, H_pad - H), (0, C_pad - C)))
    b2_p = b2.astype(jnp.float32).reshape(1, C)
    if C_pad != C:
        b2_p = jnp.pad(b2_p, ((0, 0), (0, C_pad - C)),
                       constant_values=-1e30)

    body = functools.partial(_dan_kernel, tile_b=TB, seq_len=S)

    out = pl.pallas_call(
        body,
        out_shape=jax.ShapeDtypeStruct((B_pad, C_pad), jnp.float32),
        grid_spec=pltpu.PrefetchScalarGridSpec(
            num_scalar_prefetch=1,
            grid=(B_pad // TB,),
            in_specs=[
                pl.BlockSpec(memory_space=pl.ANY),
                pl.BlockSpec((E_pad, H_pad), lambda i, ids: (0, 0)),
                pl.BlockSpec((1, H_pad), lambda i, ids: (0, 0)),
                pl.BlockSpec((H_pad, C_pad), lambda i, ids: (0, 0)),
                pl.BlockSpec((1, C_pad), lambda i, ids: (0, 0)),
            ],
            out_specs=pl.BlockSpec((TB, C_pad), lambda i, ids: (i, 0)),
            scratch_shapes=[
                pltpu.VMEM((V_pad, 1, E_pad), jnp.float32),
                pltpu.VMEM((TB, E_pad), jnp.float32),
                pltpu.SemaphoreType.DMA,
            ],
        ),
        compiler_params=pltpu.CompilerParams(
            dimension_semantics=("arbitrary",),
            vmem_limit_bytes=48 * 1024 * 1024,
        ),
    )(ids_flat, table, w1_p, b1_p, w2_p, b2_p)

    if B_pad != B or C_pad != C:
        out = out[:B, :C]
    return out


# speedup vs baseline: 1.2971x; 1.0157x over previous
import functools

import jax
import jax.numpy as jnp
from jax.experimental import pallas as pl
from jax.experimental.pallas import tpu as pltpu


def _round_up(x: int, m: int) -> int:
    return (x + m - 1) // m * m


def _dan_kernel(ids_ref,
                table_hbm,
                w1_ref,
                b1_ref,
                w2_ref,
                b2_ref,
                out_ref,
                table_ref,
                pooled_ref,
                sem,
                *, tile_b: int, seq_len: int):
    @pl.when(pl.program_id(0) == 0)
    def _():
        pltpu.make_async_copy(table_hbm, table_ref.at[:, 0], sem).start()
        pltpu.make_async_copy(table_hbm, table_ref.at[:, 0], sem).wait()

    base = pl.program_id(0) * (tile_b * seq_len)
    inv_s = jnp.float32(1.0 / seq_len)

    nacc = min(2, seq_len)

    @pl.loop(0, tile_b)
    def _(b):
        row = base + b * seq_len
        accs = [table_ref[ids_ref[row + j], 0] for j in range(nacc)]
        for s in range(nacc, seq_len):
            j = s % nacc
            accs[j] = accs[j] + table_ref[ids_ref[row + s], 0]
        while len(accs) > 1:
            accs = [a + b2 for a, b2 in zip(accs[0::2], accs[1::2])] + (
                [accs[-1]] if len(accs) % 2 else [])
        pooled_ref[b, :] = accs[0] * inv_s

    h = jnp.dot(pooled_ref[...], w1_ref[...],
                preferred_element_type=jnp.float32) + b1_ref[...]
    h = jnp.maximum(h, 0.0)

    logits = jnp.dot(h, w2_ref[...],
                     preferred_element_type=jnp.float32) + b2_ref[...]

    m = jnp.max(logits, axis=1, keepdims=True)
    lse = m + jnp.log(jnp.sum(jnp.exp(logits - m), axis=1, keepdims=True))
    out_ref[...] = logits - lse


def kernel(token_ids, emb_table, w1, b1, w2, b2):
    B, S = token_ids.shape
    V, E = emb_table.shape
    H = w1.shape[1]
    C = w2.shape[1]

    E_pad_probe = _round_up(max(E, 128), 128)
    if B * E_pad_probe * 4 <= 4 * 1024 * 1024:
        TB = _round_up(max(B, 8), 8)
    elif B >= 128:
        TB = 128
    else:
        TB = _round_up(max(B, 8), 8)
    B_pad = _round_up(B, TB)
    E_pad = _round_up(max(E, 128), 128)
    H_pad = _round_up(max(H, 128), 128)
    C_pad = _round_up(max(C, 128), 128)

    ids = token_ids.astype(jnp.int32)
    if B_pad != B:
        ids = jnp.pad(ids, ((0, B_pad - B), (0, 0)))
    ids_flat = ids.reshape(B_pad * S)

    V_pad = _round_up(V, 8)
    table = emb_table.astype(jnp.float32)
    if E_pad != E or V_pad != V:
        table = jnp.pad(table, ((0, V_pad - V), (0, E_pad - E)))

    w1_p = w1.astype(jnp.float32)
    if (E_pad, H_pad) != (E, H):
        w1_p = jnp.pad(w1_p, ((0, E_pad - E), (0, H_pad - H)))
    b1_p = b1.astype(jnp.float32).reshape(1, H)
    if H_pad != H:
        b1_p = jnp.pad(b1_p, ((0, 0), (0, H_pad - H)))
    w2_p = w2.astype(jnp.float32)
    if (H_pad, C_pad) != (H, C):
        w2_p = jnp.pad(w2_p, ((0, H_pad - H), (0, C_pad - C)))
    b2_p = b2.astype(jnp.float32).reshape(1, C)
    if C_pad != C:
        b2_p = jnp.pad(b2_p, ((0, 0), (0, C_pad - C)),
                       constant_values=-1e30)

    body = functools.partial(_dan_kernel, tile_b=TB, seq_len=S)

    out = pl.pallas_call(
        body,
        out_shape=jax.ShapeDtypeStruct((B_pad, C_pad), jnp.float32),
        grid_spec=pltpu.PrefetchScalarGridSpec(
            num_scalar_prefetch=1,
            grid=(B_pad // TB,),
            in_specs=[
                pl.BlockSpec(memory_space=pl.ANY),
                pl.BlockSpec((E_pad, H_pad), lambda i, ids: (0, 0)),
                pl.BlockSpec((1, H_pad), lambda i, ids: (0, 0)),
                pl.BlockSpec((H_pad, C_pad), lambda i, ids: (0, 0)),
                pl.BlockSpec((1, C_pad), lambda i, ids: (0, 0)),
            ],
            out_specs=pl.BlockSpec((TB, C_pad), lambda i, ids: (i, 0)),
            scratch_shapes=[
                pltpu.VMEM((V_pad, 1, E_pad), jnp.float32),
                pltpu.VMEM((TB, E_pad), jnp.float32),
                pltpu.SemaphoreType.DMA,
            ],
        ),
        compiler_params=pltpu.CompilerParams(
            dimension_semantics=("arbitrary",),
            vmem_limit_bytes=48 * 1024 * 1024,
        ),
    )(ids_flat, table, w1_p, b1_p, w2_p, b2_p)

    if B_pad != B or C_pad != C:
        out = out[:B, :C]
    return out


# speedup vs baseline: 1.3220x; 1.0192x over previous
import functools

import jax
import jax.numpy as jnp
from jax.experimental import pallas as pl
from jax.experimental.pallas import tpu as pltpu


def _round_up(x: int, m: int) -> int:
    return (x + m - 1) // m * m


def _dan_kernel(ids_ref,
                table_hbm,
                w1_ref,
                b1_ref,
                w2_ref,
                b2_ref,
                out_ref,
                table_ref,
                pooled_ref,
                sem,
                *, tile_b: int, seq_len: int):
    @pl.when(pl.program_id(0) == 0)
    def _():
        pltpu.make_async_copy(table_hbm, table_ref.at[:, 0], sem).start()
        pltpu.make_async_copy(table_hbm, table_ref.at[:, 0], sem).wait()

    base = pl.program_id(0) * (tile_b * seq_len)
    inv_s = jnp.float32(1.0 / seq_len)

    nacc = min(2, seq_len)
    nrow = 2 if tile_b % 2 == 0 else 1

    @pl.loop(0, tile_b // nrow)
    def _(bb):
        b = bb * nrow
        for r in range(nrow):
            row = base + (b + r) * seq_len
            accs = [table_ref[ids_ref[row + j], 0] for j in range(nacc)]
            for s in range(nacc, seq_len):
                j = s % nacc
                accs[j] = accs[j] + table_ref[ids_ref[row + s], 0]
            while len(accs) > 1:
                accs = [a + b2 for a, b2 in zip(accs[0::2], accs[1::2])] + (
                    [accs[-1]] if len(accs) % 2 else [])
            pooled_ref[b + r, :] = accs[0] * inv_s

    h = jnp.dot(pooled_ref[...], w1_ref[...],
                preferred_element_type=jnp.float32) + b1_ref[...]
    h = jnp.maximum(h, 0.0)

    logits = jnp.dot(h, w2_ref[...],
                     preferred_element_type=jnp.float32) + b2_ref[...]

    m = jnp.max(logits, axis=1, keepdims=True)
    lse = m + jnp.log(jnp.sum(jnp.exp(logits - m), axis=1, keepdims=True))
    out_ref[...] = logits - lse


def kernel(token_ids, emb_table, w1, b1, w2, b2):
    B, S = token_ids.shape
    V, E = emb_table.shape
    H = w1.shape[1]
    C = w2.shape[1]

    E_pad_probe = _round_up(max(E, 128), 128)
    if B * E_pad_probe * 4 <= 4 * 1024 * 1024:
        TB = _round_up(max(B, 8), 8)
    elif B >= 128:
        TB = 128
    else:
        TB = _round_up(max(B, 8), 8)
    B_pad = _round_up(B, TB)
    E_pad = _round_up(max(E, 128), 128)
    H_pad = _round_up(max(H, 128), 128)
    C_pad = _round_up(max(C, 128), 128)

    ids = token_ids.astype(jnp.int32)
    if B_pad != B:
        ids = jnp.pad(ids, ((0, B_pad - B), (0, 0)))
    ids_flat = ids.reshape(B_pad * S)

    V_pad = _round_up(V, 8)
    table = emb_table.astype(jnp.float32)
    if E_pad != E or V_pad != V:
        table = jnp.pad(table, ((0, V_pad - V), (0, E_pad - E)))

    w1_p = w1.astype(jnp.float32)
    if (E_pad, H_pad) != (E, H):
        w1_p = jnp.pad(w1_p, ((0, E_pad - E), (0, H_pad - H)))
    b1_p = b1.astype(jnp.float32).reshape(1, H)
    if H_pad != H:
        b1_p = jnp.pad(b1_p, ((0, 0), (0, H_pad - H)))
    w2_p = w2.astype(jnp.float32)
    if (H_pad, C_pad) != (H, C):
        w2_p = jnp.pad(w2_p, ((0, H_pad - H), (0, C_pad - C)))
    b2_p = b2.astype(jnp.float32).reshape(1, C)
    if C_pad != C:
        b2_p = jnp.pad(b2_p, ((0, 0), (0, C_pad - C)),
                       constant_values=-1e30)

    body = functools.partial(_dan_kernel, tile_b=TB, seq_len=S)

    out = pl.pallas_call(
        body,
        out_shape=jax.ShapeDtypeStruct((B_pad, C_pad), jnp.float32),
        grid_spec=pltpu.PrefetchScalarGridSpec(
            num_scalar_prefetch=1,
            grid=(B_pad // TB,),
            in_specs=[
                pl.BlockSpec(memory_space=pl.ANY),
                pl.BlockSpec((E_pad, H_pad), lambda i, ids: (0, 0)),
                pl.BlockSpec((1, H_pad), lambda i, ids: (0, 0)),
                pl.BlockSpec((H_pad, C_pad), lambda i, ids: (0, 0)),
                pl.BlockSpec((1, C_pad), lambda i, ids: (0, 0)),
            ],
            out_specs=pl.BlockSpec((TB, C_pad), lambda i, ids: (i, 0)),
            scratch_shapes=[
                pltpu.VMEM((V_pad, 1, E_pad), jnp.float32),
                pltpu.VMEM((TB, E_pad), jnp.float32),
                pltpu.SemaphoreType.DMA,
            ],
        ),
        compiler_params=pltpu.CompilerParams(
            dimension_semantics=("arbitrary",),
            vmem_limit_bytes=48 * 1024 * 1024,
        ),
    )(ids_flat, table, w1_p, b1_p, w2_p, b2_p)

    if B_pad != B or C_pad != C:
        out = out[:B, :C]
    return out


# speedup vs baseline: 1.3321x; 1.0076x over previous
import functools

import jax
import jax.numpy as jnp
from jax.experimental import pallas as pl
from jax.experimental.pallas import tpu as pltpu


def _round_up(x: int, m: int) -> int:
    return (x + m - 1) // m * m


def _dan_kernel(ids_ref,
                table_hbm,
                w1_ref,
                b1_ref,
                w2_ref,
                b2_ref,
                out_ref,
                table_ref,
                pooled_ref,
                sem,
                *, tile_b: int, seq_len: int):
    @pl.when(pl.program_id(0) == 0)
    def _():
        pltpu.make_async_copy(table_hbm, table_ref.at[:, 0], sem).start()
        pltpu.make_async_copy(table_hbm, table_ref.at[:, 0], sem).wait()

    base = pl.program_id(0) * (tile_b * seq_len)
    inv_s = jnp.float32(1.0 / seq_len)

    nacc = min(2, seq_len)
    nrow = 4 if tile_b % 4 == 0 else (2 if tile_b % 2 == 0 else 1)

    @pl.loop(0, tile_b // nrow)
    def _(bb):
        b = bb * nrow
        for r in range(nrow):
            row = base + (b + r) * seq_len
            accs = [table_ref[ids_ref[row + j], 0] for j in range(nacc)]
            for s in range(nacc, seq_len):
                j = s % nacc
                accs[j] = accs[j] + table_ref[ids_ref[row + s], 0]
            while len(accs) > 1:
                accs = [a + b2 for a, b2 in zip(accs[0::2], accs[1::2])] + (
                    [accs[-1]] if len(accs) % 2 else [])
            pooled_ref[b + r, :] = accs[0] * inv_s

    h = jnp.dot(pooled_ref[...], w1_ref[...],
                preferred_element_type=jnp.float32) + b1_ref[...]
    h = jnp.maximum(h, 0.0)

    logits = jnp.dot(h, w2_ref[...],
                     preferred_element_type=jnp.float32) + b2_ref[...]

    m = jnp.max(logits, axis=1, keepdims=True)
    lse = m + jnp.log(jnp.sum(jnp.exp(logits - m), axis=1, keepdims=True))
    out_ref[...] = logits - lse


def kernel(token_ids, emb_table, w1, b1, w2, b2):
    B, S = token_ids.shape
    V, E = emb_table.shape
    H = w1.shape[1]
    C = w2.shape[1]

    E_pad_probe = _round_up(max(E, 128), 128)
    if B * E_pad_probe * 4 <= 4 * 1024 * 1024:
        TB = _round_up(max(B, 8), 8)
    elif B >= 128:
        TB = 128
    else:
        TB = _round_up(max(B, 8), 8)
    B_pad = _round_up(B, TB)
    E_pad = _round_up(max(E, 128), 128)
    H_pad = _round_up(max(H, 128), 128)
    C_pad = _round_up(max(C, 128), 128)

    ids = token_ids.astype(jnp.int32)
    if B_pad != B:
        ids = jnp.pad(ids, ((0, B_pad - B), (0, 0)))
    ids_flat = ids.reshape(B_pad * S)

    V_pad = _round_up(V, 8)
    table = emb_table.astype(jnp.float32)
    if E_pad != E or V_pad != V:
        table = jnp.pad(table, ((0, V_pad - V), (0, E_pad - E)))

    w1_p = w1.astype(jnp.float32)
    if (E_pad, H_pad) != (E, H):
        w1_p = jnp.pad(w1_p, ((0, E_pad - E), (0, H_pad - H)))
    b1_p = b1.astype(jnp.float32).reshape(1, H)
    if H_pad != H:
        b1_p = jnp.pad(b1_p, ((0, 0), (0, H_pad - H)))
    w2_p = w2.astype(jnp.float32)
    if (H_pad, C_pad) != (H, C):
        w2_p = jnp.pad(w2_p, ((0, H_pad - H), (0, C_pad - C)))
    b2_p = b2.astype(jnp.float32).reshape(1, C)
    if C_pad != C:
        b2_p = jnp.pad(b2_p, ((0, 0), (0, C_pad - C)),
                       constant_values=-1e30)

    body = functools.partial(_dan_kernel, tile_b=TB, seq_len=S)

    out = pl.pallas_call(
        body,
        out_shape=jax.ShapeDtypeStruct((B_pad, C_pad), jnp.float32),
        grid_spec=pltpu.PrefetchScalarGridSpec(
            num_scalar_prefetch=1,
            grid=(B_pad // TB,),
            in_specs=[
                pl.BlockSpec(memory_space=pl.ANY),
                pl.BlockSpec((E_pad, H_pad), lambda i, ids: (0, 0)),
                pl.BlockSpec((1, H_pad), lambda i, ids: (0, 0)),
                pl.BlockSpec((H_pad, C_pad), lambda i, ids: (0, 0)),
                pl.BlockSpec((1, C_pad), lambda i, ids: (0, 0)),
            ],
            out_specs=pl.BlockSpec((TB, C_pad), lambda i, ids: (i, 0)),
            scratch_shapes=[
                pltpu.VMEM((V_pad, 1, E_pad), jnp.float32),
                pltpu.VMEM((TB, E_pad), jnp.float32),
                pltpu.SemaphoreType.DMA,
            ],
        ),
        compiler_params=pltpu.CompilerParams(
            dimension_semantics=("arbitrary",),
            vmem_limit_bytes=48 * 1024 * 1024,
        ),
    )(ids_flat, table, w1_p, b1_p, w2_p, b2_p)

    if B_pad != B or C_pad != C:
        out = out[:B, :C]
    return out


# speedup vs baseline: 1.3369x; 1.0036x over previous
import functools

import jax
import jax.numpy as jnp
from jax.experimental import pallas as pl
from jax.experimental.pallas import tpu as pltpu


def _round_up(x: int, m: int) -> int:
    return (x + m - 1) // m * m


def _dan_kernel(ids_ref,
                table_hbm,
                w1_ref,
                b1_ref,
                w2_ref,
                b2_ref,
                out_ref,
                table_ref,
                pooled_ref,
                sem,
                *, tile_b: int, seq_len: int):
    @pl.when(pl.program_id(0) == 0)
    def _():
        pltpu.make_async_copy(table_hbm, table_ref.at[:, 0], sem).start()
        pltpu.make_async_copy(table_hbm, table_ref.at[:, 0], sem).wait()

    base = pl.program_id(0) * (tile_b * seq_len)
    inv_s = jnp.float32(1.0 / seq_len)

    nacc = min(2, seq_len)
    nrow = 8 if tile_b % 8 == 0 else (2 if tile_b % 2 == 0 else 1)

    @pl.loop(0, tile_b // nrow)
    def _(bb):
        b = bb * nrow
        for r in range(nrow):
            row = base + (b + r) * seq_len
            accs = [table_ref[ids_ref[row + j], 0] for j in range(nacc)]
            for s in range(nacc, seq_len):
                j = s % nacc
                accs[j] = accs[j] + table_ref[ids_ref[row + s], 0]
            while len(accs) > 1:
                accs = [a + b2 for a, b2 in zip(accs[0::2], accs[1::2])] + (
                    [accs[-1]] if len(accs) % 2 else [])
            pooled_ref[b + r, :] = accs[0] * inv_s

    h = jnp.dot(pooled_ref[...], w1_ref[...],
                preferred_element_type=jnp.float32) + b1_ref[...]
    h = jnp.maximum(h, 0.0)

    logits = jnp.dot(h, w2_ref[...],
                     preferred_element_type=jnp.float32) + b2_ref[...]

    m = jnp.max(logits, axis=1, keepdims=True)
    lse = m + jnp.log(jnp.sum(jnp.exp(logits - m), axis=1, keepdims=True))
    out_ref[...] = logits - lse


def kernel(token_ids, emb_table, w1, b1, w2, b2):
    B, S = token_ids.shape
    V, E = emb_table.shape
    H = w1.shape[1]
    C = w2.shape[1]

    E_pad_probe = _round_up(max(E, 128), 128)
    if B * E_pad_probe * 4 <= 4 * 1024 * 1024:
        TB = _round_up(max(B, 8), 8)
    elif B >= 128:
        TB = 128
    else:
        TB = _round_up(max(B, 8), 8)
    B_pad = _round_up(B, TB)
    E_pad = _round_up(max(E, 128), 128)
    H_pad = _round_up(max(H, 128), 128)
    C_pad = _round_up(max(C, 128), 128)

    ids = token_ids.astype(jnp.int32)
    if B_pad != B:
        ids = jnp.pad(ids, ((0, B_pad - B), (0, 0)))
    ids_flat = ids.reshape(B_pad * S)

    V_pad = _round_up(V, 8)
    table = emb_table.astype(jnp.float32)
    if E_pad != E or V_pad != V:
        table = jnp.pad(table, ((0, V_pad - V), (0, E_pad - E)))

    w1_p = w1.astype(jnp.float32)
    if (E_pad, H_pad) != (E, H):
        w1_p = jnp.pad(w1_p, ((0, E_pad - E), (0, H_pad - H)))
    b1_p = b1.astype(jnp.float32).reshape(1, H)
    if H_pad != H:
        b1_p = jnp.pad(b1_p, ((0, 0), (0, H_pad - H)))
    w2_p = w2.astype(jnp.float32)
    if (H_pad, C_pad) != (H, C):
        w2_p = jnp.pad(w2_p, ((0, H_pad - H), (0, C_pad - C)))
    b2_p = b2.astype(jnp.float32).reshape(1, C)
    if C_pad != C:
        b2_p = jnp.pad(b2_p, ((0, 0), (0, C_pad - C)),
                       constant_values=-1e30)

    body = functools.partial(_dan_kernel, tile_b=TB, seq_len=S)

    out = pl.pallas_call(
        body,
        out_shape=jax.ShapeDtypeStruct((B_pad, C_pad), jnp.float32),
        grid_spec=pltpu.PrefetchScalarGridSpec(
            num_scalar_prefetch=1,
            grid=(B_pad // TB,),
            in_specs=[
                pl.BlockSpec(memory_space=pl.ANY),
                pl.BlockSpec((E_pad, H_pad), lambda i, ids: (0, 0)),
                pl.BlockSpec((1, H_pad), lambda i, ids: (0, 0)),
                pl.BlockSpec((H_pad, C_pad), lambda i, ids: (0, 0)),
                pl.BlockSpec((1, C_pad), lambda i, ids: (0, 0)),
            ],
            out_specs=pl.BlockSpec((TB, C_pad), lambda i, ids: (i, 0)),
            scratch_shapes=[
                pltpu.VMEM((V_pad, 1, E_pad), jnp.float32),
                pltpu.VMEM((TB, E_pad), jnp.float32),
                pltpu.SemaphoreType.DMA,
            ],
        ),
        compiler_params=pltpu.CompilerParams(
            dimension_semantics=("arbitrary",),
            vmem_limit_bytes=48 * 1024 * 1024,
        ),
    )(ids_flat, table, w1_p, b1_p, w2_p, b2_p)

    if B_pad != B or C_pad != C:
        out = out[:B, :C]
    return out
